# Initial kernel scaffold; baseline (speedup 1.0000x reference)
import jax
import jax.numpy as jnp
from jax.experimental import pallas as pl


def kernel(inputs, embeddings):
    raise NotImplementedError("write your pallas kernel here")


# speedup vs baseline: 1.1042x; 1.1042x over previous
import functools

import jax
import jax.numpy as jnp
from jax import lax
from jax.experimental import pallas as pl
from jax.experimental.pallas import tpu as pltpu
from jax.experimental.pallas import tpu_sc as plsc

VOCAB = 1000000
D = 32
SCALE = D ** 0.5

NC = 2
NS = 16
NW = NC * NS

C = 128
N_TOK = 16384 * 50
B_PER_W = N_TOK // NW
CHUNKS = B_PER_W // C


def _body(idx_hbm, table_hbm, out_hbm, idx_v, rows_v, sem):
    wid = lax.axis_index("s") * NC + lax.axis_index("c")
    pltpu.sync_copy(idx_hbm.at[wid], idx_v)

    def chunk(j, carry):
        pltpu.async_copy(table_hbm.at[idx_v.at[j]], rows_v, sem).wait()

        def scale_row(r, carry2):
            a = rows_v[r, pl.ds(0, 16)]
            rows_v[r, pl.ds(0, 16)] = a * SCALE
            b = rows_v[r, pl.ds(16, 16)]
            rows_v[r, pl.ds(16, 16)] = b * SCALE
            return carry2

        lax.fori_loop(0, C, scale_row, 0, unroll=4)
        pltpu.sync_copy(rows_v, out_hbm.at[wid, j])
        return carry

    lax.fori_loop(0, CHUNKS, chunk, 0)


@functools.partial(jax.jit, static_argnums=())
def _lookup(idx, table):
    mesh = plsc.VectorSubcoreMesh(core_axis_name="c", subcore_axis_name="s")
    k = pl.kernel(
        _body,
        out_type=jax.ShapeDtypeStruct((NW, CHUNKS, C, D), jnp.float32),
        mesh=mesh,
        scratch_types=[
            pltpu.VMEM((CHUNKS, C), jnp.int32),
            pltpu.VMEM((C, D), jnp.float32),
            pltpu.SemaphoreType.DMA,
        ],
        compiler_params=pltpu.CompilerParams(use_tc_tiling_on_sc=False),
    )
    return k(idx, table)


def kernel(inputs, embeddings):
    idx = inputs.astype(jnp.int32).reshape(NW, CHUNKS, C)
    out = _lookup(idx, embeddings)
    return out.reshape(inputs.shape[0], inputs.shape[1], D)


# speedup vs baseline: 1.1069x; 1.0025x over previous
import functools

import jax
import jax.numpy as jnp
from jax import lax
from jax.experimental import pallas as pl
from jax.experimental.pallas import tpu as pltpu
from jax.experimental.pallas import tpu_sc as plsc

VOCAB = 1000000
D = 32
SCALE = D ** 0.5

NC = 2
NS = 16
NW = NC * NS

C = 128
NBUF = 4
N_TOK = 16384 * 50
B_PER_W = N_TOK // NW
CHUNKS = B_PER_W // C
ROUNDS = CHUNKS // NBUF


def _body(idx_hbm, table_hbm, out_hbm, idx_v, *rest):
    in_bufs = rest[0:NBUF]
    out_bufs = rest[NBUF:2 * NBUF]
    sem_in = rest[2 * NBUF:3 * NBUF]
    sem_out = rest[3 * NBUF:4 * NBUF]

    wid = lax.axis_index("s") * NC + lax.axis_index("c")
    pltpu.sync_copy(idx_hbm.at[wid], idx_v)

    for b in range(NBUF):
        pltpu.make_async_copy(
            table_hbm.at[idx_v.at[b]], in_bufs[b], sem_in[b]).start()

    def round_fn(r, carry):
        for b in range(NBUF):
            j = r * NBUF + b
            jn = jnp.minimum(j + NBUF, CHUNKS - 1)
            pltpu.make_async_copy(
                table_hbm.at[idx_v.at[j]], in_bufs[b], sem_in[b]).wait()
            out_desc = pltpu.make_async_copy(
                out_bufs[b], out_hbm.at[wid, j], sem_out[b])

            @pl.when(r > 0)
            def _wait_prev():
                out_desc.wait()

            def scale_row(row, carry2):
                out_bufs[b][row, pl.ds(0, 16)] = (
                    in_bufs[b][row, pl.ds(0, 16)] * SCALE)
                out_bufs[b][row, pl.ds(16, 16)] = (
                    in_bufs[b][row, pl.ds(16, 16)] * SCALE)
                return carry2

            lax.fori_loop(0, C, scale_row, 0, unroll=8)

            @pl.when(j + NBUF < CHUNKS)
            def _refill():
                pltpu.make_async_copy(
                    table_hbm.at[idx_v.at[jn]], in_bufs[b], sem_in[b]).start()

            out_desc.start()
        return carry

    lax.fori_loop(0, ROUNDS, round_fn, 0)

    for b in range(NBUF):
        pltpu.make_async_copy(
            out_bufs[b], out_hbm.at[wid, CHUNKS - NBUF + b],
            sem_out[b]).wait()


@functools.partial(jax.jit, static_argnums=())
def _lookup(idx, table):
    mesh = plsc.VectorSubcoreMesh(core_axis_name="c", subcore_axis_name="s")
    scratch = [pltpu.VMEM((CHUNKS, C), jnp.int32)]
    scratch += [pltpu.VMEM((C, D), jnp.float32) for _ in range(2 * NBUF)]
    scratch += [pltpu.SemaphoreType.DMA for _ in range(2 * NBUF)]
    k = pl.kernel(
        _body,
        out_type=jax.ShapeDtypeStruct((NW, CHUNKS, C, D), jnp.float32),
        mesh=mesh,
        scratch_types=scratch,
        compiler_params=pltpu.CompilerParams(use_tc_tiling_on_sc=False),
    )
    return k(idx, table)


def kernel(inputs, embeddings):
    idx = inputs.astype(jnp.int32).reshape(NW, CHUNKS, C)
    out = _lookup(idx, embeddings)
    return out.reshape(inputs.shape[0], inputs.shape[1], D)


# speedup vs baseline: 1.3545x; 1.2237x over previous
import functools

import jax
import jax.numpy as jnp
from jax import lax
from jax.experimental import pallas as pl
from jax.experimental.pallas import tpu as pltpu
from jax.experimental.pallas import tpu_sc as plsc

VOCAB = 1000000
D = 32
SCALE = D ** 0.5

NC = 2
NS = 16
NW = NC * NS

N_BATCH = 16384
N_SLOT = 50
C = 128
N_PER_W = N_BATCH // NW
CPS = N_PER_W // C
CHUNKS = N_SLOT * CPS


def _body(idx_hbm, table_hbm, out_hbm, idx_v, *rest):
    rows_b = rest[0:CPS]
    trans_b = rest[CPS:2 * CPS]
    sem_in = rest[2 * CPS:3 * CPS]
    sem_out = rest[3 * CPS:4 * CPS]

    wid = lax.axis_index("s") * NC + lax.axis_index("c")
    n_base = wid * N_PER_W
    pltpu.sync_copy(idx_hbm.at[wid], idx_v)

    lanes = lax.iota(jnp.int32, 16)

    for c in range(CPS):
        pltpu.make_async_copy(
            table_hbm.at[idx_v.at[c]], rows_b[c], sem_in[c]).start()

    def slot_fn(s, carry):
        for c in range(CPS):
            j = s * CPS + c
            pltpu.make_async_copy(
                table_hbm.at[idx_v.at[j]], rows_b[c], sem_in[c]).wait()
            out_slice = out_hbm.at[s, :, pl.ds(n_base + c * C, C)]
            out_desc = pltpu.make_async_copy(trans_b[c], out_slice, sem_out[c])

            @pl.when(s > 0)
            def _wait_prev():
                out_desc.wait()

            def feat_fn(f, carry2):
                fcol = jnp.full((16,), 0, jnp.int32) + f
                for g in range(C // 16):
                    v = plsc.load_gather(rows_b[c], [lanes + g * 16, fcol])
                    trans_b[c][f, pl.ds(g * 16, 16)] = v * SCALE
                return carry2

            lax.fori_loop(0, D, feat_fn, 0)

            @pl.when(s + 1 < N_SLOT)
            def _refill():
                jn = jnp.minimum(j + CPS, CHUNKS - 1)
                pltpu.make_async_copy(
                    table_hbm.at[idx_v.at[jn]], rows_b[c], sem_in[c]).start()

            out_desc.start()
        return carry

    lax.fori_loop(0, N_SLOT, slot_fn, 0)

    for c in range(CPS):
        pltpu.make_async_copy(
            trans_b[c],
            out_hbm.at[N_SLOT - 1, :, pl.ds(n_base + c * C, C)],
            sem_out[c]).wait()


@functools.partial(jax.jit, static_argnums=())
def _lookup(idx, table):
    mesh = plsc.VectorSubcoreMesh(core_axis_name="c", subcore_axis_name="s")
    scratch = [pltpu.VMEM((CHUNKS, C), jnp.int32)]
    scratch += [pltpu.VMEM((C, D), jnp.float32) for _ in range(CPS)]
    scratch += [pltpu.VMEM((D, C), jnp.float32) for _ in range(CPS)]
    scratch += [pltpu.SemaphoreType.DMA for _ in range(2 * CPS)]
    k = pl.kernel(
        _body,
        out_type=jax.ShapeDtypeStruct((N_SLOT, D, N_BATCH), jnp.float32),
        mesh=mesh,
        scratch_types=scratch,
        compiler_params=pltpu.CompilerParams(
            use_tc_tiling_on_sc=False, needs_layout_passes=False),
    )
    return k(idx, table)


def kernel(inputs, embeddings):
    idx = (inputs.astype(jnp.int32).T
           .reshape(N_SLOT, NW, CPS, C)
           .transpose(1, 0, 2, 3)
           .reshape(NW, CHUNKS, C))
    out = _lookup(idx, embeddings)
    return jnp.transpose(out, (2, 0, 1))


# speedup vs baseline: 1.9356x; 1.4290x over previous
import functools

import jax
import jax.numpy as jnp
from jax import lax
from jax.experimental import pallas as pl
from jax.experimental.pallas import tpu as pltpu
from jax.experimental.pallas import tpu_sc as plsc

VOCAB = 1000000
D = 32
SCALE = D ** 0.5

NC = 2
NS = 16
NW = NC * NS

N_BATCH = 16384
N_SLOT = 50
C = 128
N_PER_W = N_BATCH // NW
CPS = N_PER_W // C
CHUNKS = N_SLOT * CPS


def _body(idx_hbm, table_hbm, out_hbm, idx_v, *rest):
    rows_b = rest[0:CPS]
    trans_b = rest[CPS:2 * CPS]
    sem_in = rest[2 * CPS:3 * CPS]
    sem_out = rest[3 * CPS:4 * CPS]

    wid = lax.axis_index("s") * NC + lax.axis_index("c")
    n_base = wid * N_PER_W
    pltpu.sync_copy(idx_hbm.at[wid], idx_v)

    lanes = lax.iota(jnp.int32, 16)

    for c in range(CPS):
        pltpu.make_async_copy(
            table_hbm.at[idx_v.at[c]],
            rows_b[c], sem_in[c]).start()

    def slot_fn(s, carry):
        for c in range(CPS):
            j = s * CPS + c
            pltpu.make_async_copy(
                table_hbm.at[idx_v.at[j]],
                rows_b[c], sem_in[c]).wait()
            out_slice = out_hbm.at[s, :, pl.ds(n_base + c * C, C)]
            out_desc = pltpu.make_async_copy(
                trans_b[c].at[:, pl.ds(0, C)], out_slice, sem_out[c])

            @pl.when(s > 0)
            def _wait_prev():
                out_desc.wait()

            def row_fn(r, carry2):
                rcol = jnp.full((16,), 0, jnp.int32) + r
                for f0 in range(0, D, 16):
                    v = rows_b[c][r, pl.ds(f0, 16)]
                    plsc.store_scatter(
                        trans_b[c], [lanes + f0, rcol], v * SCALE)
                return carry2

            lax.fori_loop(0, C, row_fn, 0)

            @pl.when(s + 1 < N_SLOT)
            def _refill():
                jn = jnp.minimum(j + CPS, CHUNKS - 1)
                pltpu.make_async_copy(
                    table_hbm.at[idx_v.at[jn]], rows_b[c], sem_in[c]).start()

            out_desc.start()
        return carry

    lax.fori_loop(0, N_SLOT, slot_fn, 0)

    for c in range(CPS):
        pltpu.make_async_copy(
            trans_b[c].at[:, pl.ds(0, C)],
            out_hbm.at[N_SLOT - 1, :, pl.ds(n_base + c * C, C)],
            sem_out[c]).wait()


@functools.partial(jax.jit, static_argnums=())
def _lookup(idx, table):
    mesh = plsc.VectorSubcoreMesh(core_axis_name="c", subcore_axis_name="s")
    scratch = [pltpu.VMEM((CHUNKS, C), jnp.int32)]
    scratch += [pltpu.VMEM((C, D), jnp.float32) for _ in range(CPS)]
    scratch += [pltpu.VMEM((D, C + 1), jnp.float32) for _ in range(CPS)]
    scratch += [pltpu.SemaphoreType.DMA for _ in range(2 * CPS)]
    k = pl.kernel(
        _body,
        out_type=jax.ShapeDtypeStruct((N_SLOT, D, N_BATCH), jnp.float32),
        mesh=mesh,
        scratch_types=scratch,
        compiler_params=pltpu.CompilerParams(
            use_tc_tiling_on_sc=False, needs_layout_passes=False),
    )
    return k(idx, table)


def kernel(inputs, embeddings):
    idx = (inputs.astype(jnp.int32).T
           .reshape(N_SLOT, NW, CPS, C)
           .transpose(1, 0, 2, 3)
           .reshape(NW, CHUNKS, C))
    out = _lookup(idx, embeddings)
    return jnp.transpose(out, (2, 0, 1))


# speedup vs baseline: 2.3038x; 1.1902x over previous
import functools

import jax
import jax.numpy as jnp
from jax import lax
from jax.experimental import pallas as pl
from jax.experimental.pallas import tpu as pltpu
from jax.experimental.pallas import tpu_sc as plsc

VOCAB = 1000000
D = 32
SCALE = D ** 0.5

NC = 2
NS = 16
NW = NC * NS

N_BATCH = 16384
N_SLOT = 50
C = 128
N_PER_W = N_BATCH // NW
CPS = N_PER_W // C
CHUNKS = N_SLOT * CPS


def _body(idx_hbm, table_hbm, out_hbm, idx_v, *rest):
    rows_b = rest[0:CPS]
    trans_b = rest[CPS:2 * CPS]
    sem_in = rest[2 * CPS:3 * CPS]
    sem_out = rest[3 * CPS:4 * CPS]

    wid = lax.axis_index("s") * NC + lax.axis_index("c")
    n_base = wid * N_PER_W
    pltpu.sync_copy(idx_hbm.at[wid], idx_v)

    lanes = lax.iota(jnp.int32, 16)

    for c in range(CPS):
        pltpu.make_async_copy(
            table_hbm.at[idx_v.at[c]],
            rows_b[c], sem_in[c]).start()

    def slot_fn(s, carry):
        for c in range(CPS):
            j = s * CPS + c
            pltpu.make_async_copy(
                table_hbm.at[idx_v.at[j]],
                rows_b[c], sem_in[c]).wait()
            tcol = (n_base + c * C) // C
            out_desc = [
                pltpu.make_async_copy(
                    trans_b[c].at[pl.ds(8 * k, 8), pl.ds(0, C)],
                    out_hbm.at[4 * s + k, tcol], sem_out[c])
                for k in range(4)]

            @pl.when(s > 0)
            def _wait_prev():
                for k in range(4):
                    out_desc[k].wait()

            def row_fn(r, carry2):
                rcol = jnp.full((16,), 0, jnp.int32) + r
                for f0 in range(0, D, 16):
                    v = rows_b[c][r, pl.ds(f0, 16)]
                    plsc.store_scatter(
                        trans_b[c], [lanes + f0, rcol], v * SCALE)
                return carry2

            lax.fori_loop(0, C, row_fn, 0, unroll=8)

            @pl.when(s + 1 < N_SLOT)
            def _refill():
                jn = jnp.minimum(j + CPS, CHUNKS - 1)
                pltpu.make_async_copy(
                    table_hbm.at[idx_v.at[jn]], rows_b[c], sem_in[c]).start()

            for k in range(4):
                out_desc[k].start()
        return carry

    lax.fori_loop(0, N_SLOT, slot_fn, 0)

    for c in range(CPS):
        for k in range(4):
            pltpu.make_async_copy(
                trans_b[c].at[pl.ds(8 * k, 8), pl.ds(0, C)],
                out_hbm.at[4 * (N_SLOT - 1) + k, (n_base + c * C) // C],
                sem_out[c]).wait()


@functools.partial(jax.jit, static_argnums=())
def _lookup(idx, table):
    mesh = plsc.VectorSubcoreMesh(core_axis_name="c", subcore_axis_name="s")
    scratch = [pltpu.VMEM((CHUNKS, C), jnp.int32)]
    scratch += [pltpu.VMEM((C, D), jnp.float32) for _ in range(CPS)]
    scratch += [pltpu.VMEM((D, C + 1), jnp.float32) for _ in range(CPS)]
    scratch += [pltpu.SemaphoreType.DMA for _ in range(2 * CPS)]
    k = pl.kernel(
        _body,
        out_type=jax.ShapeDtypeStruct(
            (N_SLOT * D // 8, N_BATCH // C, 8, C), jnp.float32),
        mesh=mesh,
        scratch_types=scratch,
        compiler_params=pltpu.CompilerParams(
            use_tc_tiling_on_sc=False, needs_layout_passes=False),
    )
    return k(idx, table)


def kernel(inputs, embeddings):
    idx = (inputs.astype(jnp.int32).T
           .reshape(N_SLOT, NW, CPS, C)
           .transpose(1, 0, 2, 3)
           .reshape(NW, CHUNKS, C))
    out = _lookup(idx, embeddings)
    out = out.transpose(0, 2, 1, 3).reshape(N_SLOT, D, N_BATCH)
    return jnp.transpose(out, (2, 0, 1))
